# baseline (speedup 1.0000x reference)
import functools

import jax
import jax.numpy as jnp
from jax import lax
from jax.experimental import pallas as pl
from jax.experimental.pallas import tpu as pltpu
from jax.experimental.pallas import tpu_sc as plsc

_B = 16384
_D = 64
_NW = 32
_BPW = _B // _NW
_CH = 128
_NCH = _BPW // _CH
_L = 16


@functools.partial(
    pl.kernel,
    out_type=jax.ShapeDtypeStruct((_NW, _L), jnp.float32),
    mesh=plsc.VectorSubcoreMesh(core_axis_name="c", subcore_axis_name="s"),
    scratch_types=[
        pltpu.VMEM((_NCH, _CH), jnp.int32),
        pltpu.VMEM((_NCH, _CH, _D), jnp.float32),
        pltpu.VMEM((_NCH, _CH, _D), jnp.float32),
        pltpu.VMEM((_L,), jnp.float32),
        pltpu.SemaphoreType.DMA,
    ],
    compiler_params=pltpu.CompilerParams(use_tc_tiling_on_sc=False),
)
def _center_loss_sc(feat_hbm, lab_hbm, cent_hbm, out_hbm,
                    idx_v, rows_v, feat_v, acc_v, sem):
    wid = lax.axis_index("s") * 2 + lax.axis_index("c")

    pltpu.sync_copy(lab_hbm.at[wid], idx_v)
    handles = [
        pltpu.async_copy(cent_hbm.at[idx_v.at[j]], rows_v.at[j], sem)
        for j in range(_NCH)
    ]
    pltpu.sync_copy(feat_hbm.at[wid], feat_v)

    acc = jnp.zeros((_L,), jnp.float32)
    for j in range(_NCH):
        handles[j].wait()

        def body(k, a, j=j):
            for ci in range(_D // _L):
                f = feat_v[j, k, pl.ds(ci * _L, _L)]
                c = rows_v[j, k, pl.ds(ci * _L, _L)]
                d = f - c
                a = a + d * d
            return a

        acc = lax.fori_loop(0, _CH, body, acc)

    acc_v[...] = acc
    pltpu.sync_copy(acc_v, out_hbm.at[wid])


def kernel(features, labels, centers):
    batch = features.shape[0]
    feat_r = features.reshape(_NW, _NCH, _CH, _D)
    lab_r = labels.astype(jnp.int32).reshape(_NW, _NCH, _CH)
    partials = _center_loss_sc(feat_r, lab_r, centers)
    return jnp.sum(partials) / 2.0 / batch
